# Initial kernel scaffold; baseline (speedup 1.0000x reference)
import functools

import jax
import jax.numpy as jnp
from jax import lax
from jax.experimental import pallas as pl
from jax.experimental.pallas import tpu as pltpu
from jax.experimental.pallas import tpu_sc as plsc

N = 10000
D = 128
NP = 10240
NC, NS = 2, 16
NW = NC * NS
RT = NP // NS
E = 320000
CHUNK = 128
EPT = 10112
EP = EPT * NW
NCHUNK = EPT // CHUNK
DD = 16

_mesh = plsc.VectorSubcoreMesh(core_axis_name="c", subcore_axis_name="s")


@functools.partial(
    pl.kernel,
    out_type=jax.ShapeDtypeStruct((NC, NP, DD), jnp.float32),
    mesh=_mesh,
    scratch_types=[
        pltpu.VMEM((CHUNK,), jnp.int32),
        pltpu.VMEM((CHUNK, DD), jnp.float32),
        pltpu.VMEM_SHARED((NP, DD), jnp.float32),
    ],
)
def _sc_degree(ei_hbm, z_hbm, ones_hbm, out_hbm, cbuf, ones_v, acc):
    cid = lax.axis_index("c")
    sid = lax.axis_index("s")
    wid = cid * NS + sid
    pltpu.sync_copy(z_hbm, acc.at[pl.ds(sid * RT, RT)])
    pltpu.sync_copy(ones_hbm, ones_v)
    plsc.subcore_barrier()
    base0 = wid * EPT

    def step(i, carry):
        pltpu.sync_copy(ei_hbm.at[1, pl.ds(base0 + i * CHUNK, CHUNK)], cbuf)
        pltpu.sync_copy(ones_v, acc.at[cbuf], add=True)
        return carry

    lax.fori_loop(0, NCHUNK, step, 0)
    plsc.subcore_barrier()
    pltpu.sync_copy(acc.at[pl.ds(sid * RT, RT)],
                    out_hbm.at[cid, pl.ds(sid * RT, RT)])


@functools.partial(
    pl.kernel,
    out_type=jax.ShapeDtypeStruct((NC, NP, D), jnp.float32),
    mesh=_mesh,
    scratch_types=[
        pltpu.VMEM((2, CHUNK), jnp.int32),
        pltpu.VMEM((CHUNK, D), jnp.float32),
        pltpu.VMEM_SHARED((NP, D), jnp.float32),
        pltpu.SemaphoreType.DMA,
    ],
)
def _sc_scatter(g_hbm, ei_hbm, z_hbm, out_hbm, ebuf, msg, acc, sem):
    cid = lax.axis_index("c")
    sid = lax.axis_index("s")
    wid = cid * NS + sid
    pltpu.sync_copy(z_hbm, acc.at[pl.ds(sid * RT, RT)])
    plsc.subcore_barrier()
    base0 = wid * EPT

    def step(i, carry):
        pltpu.sync_copy(ei_hbm.at[:, pl.ds(base0 + i * CHUNK, CHUNK)], ebuf)
        pltpu.async_copy(g_hbm.at[ebuf.at[0]], msg, sem).wait()
        pltpu.sync_copy(msg, acc.at[ebuf.at[1]], add=True)
        return carry

    lax.fori_loop(0, NCHUNK, step, 0)
    plsc.subcore_barrier()
    pltpu.sync_copy(acc.at[pl.ds(sid * RT, RT)],
                    out_hbm.at[cid, pl.ds(sid * RT, RT)])


def _dinv(degp_ref):
    deg = degp_ref[0][:, 0:1] + degp_ref[1][:, 0:1] + 1.0
    mask = lax.broadcasted_iota(jnp.int32, (NP, 1), 0) < N
    return jnp.where(mask, lax.rsqrt(deg), 0.0)


def _tc_first(x_ref, w_ref, degp_ref, g_ref):
    dinv = _dinv(degp_ref)
    hw = jnp.dot(x_ref[...], w_ref[...], preferred_element_type=jnp.float32)
    g_ref[...] = hw * dinv


def _tc_mid(acc_ref, g_ref, degp_ref, w_ref, b_ref, out_ref):
    dinv = _dinv(degp_ref)
    h = jnp.maximum(dinv * (acc_ref[0] + acc_ref[1] + g_ref[...]) + b_ref[...], 0.0)
    out_ref[...] = jnp.dot(h, w_ref[...], preferred_element_type=jnp.float32) * dinv


def _tc_final(acc_ref, g_ref, degp_ref, b_ref, wf1_ref, bf1_ref, wf2_ref,
              bf2_ref, wf3_ref, bf3_ref, out_ref):
    dinv = _dinv(degp_ref)
    h = jnp.maximum(dinv * (acc_ref[0] + acc_ref[1] + g_ref[...]) + b_ref[...], 0.0)
    mask = lax.broadcasted_iota(jnp.int32, (NP, 1), 0) < N
    h = jnp.where(mask, h, 0.0)
    pooled = jnp.sum(h, axis=0, keepdims=True) * (1.0 / N)
    o = jnp.maximum(jnp.dot(pooled, wf1_ref[...],
                            preferred_element_type=jnp.float32) + bf1_ref[...], 0.0)
    o = jnp.maximum(jnp.dot(o, wf2_ref[...],
                            preferred_element_type=jnp.float32) + bf2_ref[...], 0.0)
    out_ref[...] = jnp.dot(o, wf3_ref[...],
                           preferred_element_type=jnp.float32) + bf3_ref[...]


def _pc(body, out_shape):
    return pl.pallas_call(body, out_shape=jax.ShapeDtypeStruct(out_shape, jnp.float32))


def kernel(x, edge_index, W1, b1, W2, b2, W3, b3, Wf1, bf1, Wf2, bf2, Wf3, bf3):
    x_pad = jnp.pad(x, ((0, NP - N), (0, 0)))
    ei_pad = jnp.pad(edge_index, ((0, 0), (0, EP - E)), constant_values=N)
    z = jnp.zeros((RT, D), jnp.float32)
    zd = jnp.zeros((RT, DD), jnp.float32)
    ones = jnp.zeros((CHUNK, DD), jnp.float32).at[:, 0].set(1.0)

    degp = _sc_degree(ei_pad, zd, ones)
    g1 = _pc(_tc_first, (NP, D))(x_pad, W1, degp)
    acc1 = _sc_scatter(g1, ei_pad, z)
    g2 = _pc(_tc_mid, (NP, D))(acc1, g1, degp, W2, b1)
    acc2 = _sc_scatter(g2, ei_pad, z)
    g3 = _pc(_tc_mid, (NP, D))(acc2, g2, degp, W3, b2)
    acc3 = _sc_scatter(g3, ei_pad, z)
    return _pc(_tc_final, (1, 10))(acc3, g3, degp, b3, Wf1, bf1, Wf2, bf2, Wf3, bf3)


# speedup vs baseline: 9.5883x; 9.5883x over previous
import functools

import jax
import jax.numpy as jnp
from jax import lax
from jax.experimental import pallas as pl
from jax.experimental.pallas import tpu as pltpu
from jax.experimental.pallas import tpu_sc as plsc

N = 10000
D = 128
NP = 10240
NC, NS = 2, 16
NW = NC * NS
RT = NP // NS
E = 320000
CHUNK = 128
EPT = 10112
EP = EPT * NW
NCHUNK = EPT // CHUNK
DD = 16

_mesh = plsc.VectorSubcoreMesh(core_axis_name="c", subcore_axis_name="s",
                               num_cores=NC, num_subcores=NS)


_DEG_KW = dict(
    out_type=jax.ShapeDtypeStruct((NC, NP, DD), jnp.float32),
    mesh=_mesh,
    scratch_types=[
        pltpu.VMEM((2, CHUNK), jnp.int32),
        pltpu.VMEM((CHUNK, DD), jnp.float32),
        pltpu.VMEM_SHARED((NP, DD), jnp.float32),
    ],
)


def _sc_degree_body(ei_hbm, z_hbm, ones_hbm, out_hbm, cbuf, ones_v, acc):
    cid = lax.axis_index("c")
    sid = lax.axis_index("s")
    wid = cid * NS + sid
    pltpu.sync_copy(z_hbm, acc.at[pl.ds(sid * RT, RT)])
    pltpu.sync_copy(ones_hbm, ones_v)
    plsc.subcore_barrier()
    base0 = wid * EPT

    def step(i, carry):
        pltpu.sync_copy(ei_hbm.at[:, pl.ds(base0 + i * CHUNK, CHUNK)], cbuf)
        pltpu.sync_copy(ones_v, acc.at[cbuf.at[1]], add=True)
        return carry

    lax.fori_loop(0, NCHUNK, step, 0)
    plsc.subcore_barrier()
    pltpu.sync_copy(acc.at[pl.ds(sid * RT, RT)],
                    out_hbm.at[cid, pl.ds(sid * RT, RT)])


_SCAT_KW = dict(
    out_type=jax.ShapeDtypeStruct((NC, NP, D), jnp.float32),
    mesh=_mesh,
    scratch_types=[
        pltpu.VMEM((2, CHUNK), jnp.int32),
        pltpu.VMEM((CHUNK, D), jnp.float32),
        pltpu.VMEM_SHARED((NP, D), jnp.float32),
        pltpu.SemaphoreType.DMA,
    ],
)


def _sc_scatter_body(g_hbm, ei_hbm, z_hbm, out_hbm, ebuf, msg, acc, sem):
    cid = lax.axis_index("c")
    sid = lax.axis_index("s")
    wid = cid * NS + sid
    pltpu.sync_copy(z_hbm, acc.at[pl.ds(sid * RT, RT)])
    plsc.subcore_barrier()
    base0 = wid * EPT

    def step(i, carry):
        pltpu.sync_copy(ei_hbm.at[:, pl.ds(base0 + i * CHUNK, CHUNK)], ebuf)
        pltpu.async_copy(g_hbm.at[ebuf.at[0]], msg, sem).wait()
        pltpu.sync_copy(msg, acc.at[ebuf.at[1]], add=True)
        return carry

    lax.fori_loop(0, NCHUNK, step, 0)
    plsc.subcore_barrier()
    pltpu.sync_copy(acc.at[pl.ds(sid * RT, RT)],
                    out_hbm.at[cid, pl.ds(sid * RT, RT)])


_sc_degree = pl.kernel(_sc_degree_body, **_DEG_KW)
_sc_scatter = pl.kernel(_sc_scatter_body, **_SCAT_KW)


def _dinv(degp_ref):
    deg = degp_ref[0][:, 0:1] + degp_ref[1][:, 0:1] + 1.0
    mask = lax.broadcasted_iota(jnp.int32, (NP, 1), 0) < N
    return jnp.where(mask, lax.rsqrt(deg), 0.0)


def _tc_first(x_ref, w_ref, degp_ref, g_ref):
    dinv = _dinv(degp_ref)
    hw = jnp.dot(x_ref[...], w_ref[...], preferred_element_type=jnp.float32,
                 precision=lax.Precision.HIGHEST)
    g_ref[...] = hw * dinv


def _tc_mid(acc_ref, g_ref, degp_ref, w_ref, b_ref, out_ref):
    dinv = _dinv(degp_ref)
    h = jnp.maximum(dinv * (acc_ref[0] + acc_ref[1] + g_ref[...]) + b_ref[...], 0.0)
    out_ref[...] = jnp.dot(h, w_ref[...], preferred_element_type=jnp.float32,
                           precision=lax.Precision.HIGHEST) * dinv


def _tc_final(acc_ref, g_ref, degp_ref, b_ref, wf1_ref, bf1_ref, wf2_ref,
              bf2_ref, wf3_ref, bf3_ref, out_ref):
    dinv = _dinv(degp_ref)
    h = jnp.maximum(dinv * (acc_ref[0] + acc_ref[1] + g_ref[...]) + b_ref[...], 0.0)
    mask = lax.broadcasted_iota(jnp.int32, (NP, 1), 0) < N
    h = jnp.where(mask, h, 0.0)
    pooled = jnp.sum(h, axis=0, keepdims=True) * (1.0 / N)
    hp = lax.Precision.HIGHEST
    o = jnp.maximum(jnp.dot(pooled, wf1_ref[...], precision=hp,
                            preferred_element_type=jnp.float32) + bf1_ref[...], 0.0)
    o = jnp.maximum(jnp.dot(o, wf2_ref[...], precision=hp,
                            preferred_element_type=jnp.float32) + bf2_ref[...], 0.0)
    out_ref[...] = jnp.dot(o, wf3_ref[...], precision=hp,
                           preferred_element_type=jnp.float32) + bf3_ref[...]


def _pc(body, out_shape):
    return pl.pallas_call(body, out_shape=jax.ShapeDtypeStruct(out_shape, jnp.float32))


def kernel(x, edge_index, W1, b1, W2, b2, W3, b3, Wf1, bf1, Wf2, bf2, Wf3, bf3):
    x_pad = jnp.pad(x, ((0, NP - N), (0, 0)))
    ei_pad = jnp.pad(edge_index, ((0, 0), (0, EP - E)), constant_values=N)
    z = jnp.zeros((RT, D), jnp.float32)
    zd = jnp.zeros((RT, DD), jnp.float32)
    ones = jnp.zeros((CHUNK, DD), jnp.float32).at[:, 0].set(1.0)

    degp = _sc_degree(ei_pad, zd, ones)
    g1 = _pc(_tc_first, (NP, D))(x_pad, W1, degp)
    acc1 = _sc_scatter(g1, ei_pad, z)
    g2 = _pc(_tc_mid, (NP, D))(acc1, g1, degp, W2, b1)
    acc2 = _sc_scatter(g2, ei_pad, z)
    g3 = _pc(_tc_mid, (NP, D))(acc2, g2, degp, W3, b2)
    acc3 = _sc_scatter(g3, ei_pad, z)
    return _pc(_tc_final, (1, 10))(acc3, g3, degp, b3, Wf1, bf1, Wf2, bf2, Wf3, bf3)
